# Initial kernel scaffold; baseline (speedup 1.0000x reference)
import jax
import jax.numpy as jnp
from jax.experimental import pallas as pl


def kernel(boxes, scores, labels):
    raise NotImplementedError("write your pallas kernel here")


# speedup vs baseline: 17.4201x; 17.4201x over previous
import functools

import jax
import jax.numpy as jnp
from jax import lax
from jax.experimental import pallas as pl
from jax.experimental.pallas import tpu as pltpu

_NMS_THRESH = 0.6
_SCORE_THRESH = 0.1
_DETECTIONS = 100
_NEG = -1e9

_N = 20000
_P = 20480
_ROWS = _P // 128


def _nms_body(msc, mx1, my1, mx2, my2, gx1, gy1, gx2, gy2, gsc, goff,
              outf_ref, outk_ref, m_scr, a2_scr):
    m_scr[...] = msc[...]
    a2_scr[...] = (mx2[...] - mx1[...]) * (my2[...] - my1[...])

    lin = (lax.broadcasted_iota(jnp.int32, (_ROWS, 128), 0) * 128
           + lax.broadcasted_iota(jnp.int32, (_ROWS, 128), 1))
    lane = lax.broadcasted_iota(jnp.int32, (1, 128), 1)

    cmx1 = mx1[...]
    cmy1 = my1[...]
    cmx2 = mx2[...]
    cmy2 = my2[...]

    def step(t, carry):
        keep_acc, x1_acc, y1_acc, x2_acc, y2_acc, sc_acc = carry
        m = m_scr[...]
        mv = jnp.max(m)
        idx = jnp.min(jnp.where(m == mv, lin, jnp.int32(2**30)))
        r = idx // 128
        c = idx % 128
        lm = lane == c

        def gat(ref):
            return jnp.sum(jnp.where(lm, ref[r], 0.0))

        bx1 = gat(gx1)
        by1 = gat(gy1)
        bx2 = gat(gx2)
        by2 = gat(gy2)
        bsc = gat(gsc)
        boff = gat(goff)
        sx1 = bx1 + boff
        sy1 = by1 + boff
        sx2 = bx2 + boff
        sy2 = by2 + boff
        a1 = (sx2 - sx1) * (sy2 - sy1)

        x1 = jnp.maximum(sx1, cmx1)
        y1 = jnp.maximum(sy1, cmy1)
        x2 = jnp.minimum(sx2, cmx2)
        y2 = jnp.minimum(sy2, cmy2)
        inter = jnp.maximum(x2 - x1, 0.0) * jnp.maximum(y2 - y1, 0.0)
        iou = inter / (a1 + a2_scr[...] - inter + 1e-9)
        newm = jnp.where(iou > _NMS_THRESH, _NEG, m)
        newm = jnp.where(lin == idx, _NEG, newm)
        m_scr[...] = newm

        sel_t = lane == t
        return (jnp.where(sel_t, idx, keep_acc),
                jnp.where(sel_t, bx1, x1_acc),
                jnp.where(sel_t, by1, y1_acc),
                jnp.where(sel_t, bx2, x2_acc),
                jnp.where(sel_t, by2, y2_acc),
                jnp.where(sel_t, bsc, sc_acc))

    zf = jnp.zeros((1, 128), jnp.float32)
    zi = jnp.zeros((1, 128), jnp.int32)
    keep_acc, x1_acc, y1_acc, x2_acc, y2_acc, sc_acc = lax.fori_loop(
        0, _DETECTIONS, step, (zi, zf, zf, zf, zf, zf))

    outk_ref[...] = jnp.broadcast_to(keep_acc, (8, 128))
    outf_ref[...] = jnp.concatenate(
        [x1_acc, y1_acc, x2_acc, y2_acc, sc_acc, zf, zf, zf], axis=0)


@jax.jit
def kernel(boxes, scores, labels):
    off = labels.astype(boxes.dtype) * 4000.0
    msc = jnp.where(scores > _SCORE_THRESH, scores, _NEG)

    pad = _P - _N

    def pad1(x, val):
        return jnp.concatenate([x, jnp.full((pad,), val, x.dtype)])

    mscp = pad1(msc, _NEG).reshape(_ROWS, 128)
    offp = pad1(off, 0.0)
    b = [pad1(boxes[:, i], 0.0) for i in range(4)]
    mx = [(bi + offp).reshape(_ROWS, 128) for bi in b]
    g = [bi.reshape(_ROWS, 1, 128) for bi in b]
    gsc = pad1(scores, 0.0).reshape(_ROWS, 1, 128)
    goff = offp.reshape(_ROWS, 1, 128)

    outf, outk = pl.pallas_call(
        _nms_body,
        out_shape=[jax.ShapeDtypeStruct((8, 128), jnp.float32),
                   jax.ShapeDtypeStruct((8, 128), jnp.int32)],
        scratch_shapes=[pltpu.VMEM((_ROWS, 128), jnp.float32),
                        pltpu.VMEM((_ROWS, 128), jnp.float32)],
    )(mscp, *mx, *g, gsc, goff)

    keep = outk[0, :_DETECTIONS]
    out = jnp.stack([outf[i, :_DETECTIONS] for i in range(5)], axis=1)
    return out, keep
